# baseline (speedup 1.0000x reference)
import jax
import jax.numpy as jnp
from jax import lax
from jax.experimental import pallas as pl
from jax.experimental.pallas import tpu as pltpu
from jax.experimental.pallas import tpu_sc as plsc

NC = 2
NS = 16
NW = NC * NS

N_ATOMS = 10000
N_BONDS = 320000
MAX_NB = 32
H = 64

N_ATOMS_P = 10240
ATOMS_PER_W = N_ATOMS_P // NW
A_CHUNK_ATOMS = 4
A_CHUNKS = ATOMS_PER_W // A_CHUNK_ATOMS

N_BONDS_P = 327680
BONDS_PER_W = N_BONDS_P // NW
B_CHUNK = 128
B_CHUNKS = BONDS_PER_W // B_CHUNK

_SC_MESH = plsc.VectorSubcoreMesh(core_axis_name="c", subcore_axis_name="s")


def _worker_id():
    return lax.axis_index("s") * NC + lax.axis_index("c")


GW = 128


def _sc_gather_sum_body(m_hbm, a2b_hbm, out_hbm, idx_v, rows_v, acc_v, sem):
    wid = _worker_id()
    pltpu.sync_copy(a2b_hbm.at[wid], idx_v)

    def body(j, carry):
        pltpu.async_copy(m_hbm.at[idx_v.at[j]], rows_v, sem).wait()
        for a in range(A_CHUNK_ATOMS):
            for v in range(H // 16):
                acc = rows_v[a * MAX_NB, pl.ds(v * 16, 16)]
                for k in range(1, MAX_NB):
                    acc = acc + rows_v[a * MAX_NB + k, pl.ds(v * 16, 16)]
                acc_v[a, pl.ds(v * 16, 16)] = acc
        base = wid * ATOMS_PER_W + j * A_CHUNK_ATOMS
        pltpu.sync_copy(acc_v, out_hbm.at[pl.ds(base, A_CHUNK_ATOMS)])
        return carry

    lax.fori_loop(0, A_CHUNKS, body, 0)


def _sc_gather_sum(message, a2b_r):
    k = pl.kernel(
        _sc_gather_sum_body,
        out_type=jax.ShapeDtypeStruct((N_ATOMS_P, GW), jnp.float32),
        mesh=_SC_MESH,
        scratch_types=[
            pltpu.VMEM((A_CHUNKS, 128), jnp.int32),
            pltpu.VMEM((128, GW), jnp.float32),
            pltpu.VMEM((A_CHUNK_ATOMS, GW), jnp.float32),
            pltpu.SemaphoreType.DMA,
        ],
    )
    return k(message, a2b_r)


def _sc_bond_update_body(am_hbm, m_hbm, b2a_hbm, b2revb_hbm, t_hbm,
                         idx1_v, idx2_v, am_rows, rev_rows, t_buf, sem1, sem2):
    wid = _worker_id()
    pltpu.sync_copy(b2a_hbm.at[wid], idx1_v)
    pltpu.sync_copy(b2revb_hbm.at[wid], idx2_v)

    def body(j, carry):
        cp1 = pltpu.async_copy(am_hbm.at[idx1_v.at[j]], am_rows, sem1)
        cp2 = pltpu.async_copy(m_hbm.at[idx2_v.at[j]], rev_rows, sem2)
        cp1.wait()
        cp2.wait()
        for r in range(B_CHUNK):
            for v in range(H // 16):
                sl = pl.ds(v * 16, 16)
                t_buf[r, sl] = am_rows[r, sl] - rev_rows[r, sl]
        base = wid * BONDS_PER_W + j * B_CHUNK
        pltpu.sync_copy(t_buf, t_hbm.at[pl.ds(base, B_CHUNK)])
        return carry

    lax.fori_loop(0, B_CHUNKS, body, 0)


def _sc_bond_update(a_message, message, b2a_r, b2revb_r):
    k = pl.kernel(
        _sc_bond_update_body,
        out_type=jax.ShapeDtypeStruct((N_BONDS_P, H), jnp.float32),
        mesh=_SC_MESH,
        scratch_types=[
            pltpu.VMEM((B_CHUNKS, 128), jnp.int32),
            pltpu.VMEM((B_CHUNKS, 128), jnp.int32),
            pltpu.VMEM((B_CHUNK, GW), jnp.float32),
            pltpu.VMEM((B_CHUNK, GW), jnp.float32),
            pltpu.VMEM((B_CHUNK, H), jnp.float32),
            pltpu.SemaphoreType.DMA,
            pltpu.SemaphoreType.DMA,
        ],
    )
    return k(a_message, message, b2a_r, b2revb_r)


_BLK = 8000


def _tc_init_body(fb_ref, wi_ref, bi_ref, inp0_ref, m_ref):
    x = jnp.dot(fb_ref[...], wi_ref[...],
                preferred_element_type=jnp.float32) + bi_ref[...]
    inp0_ref[...] = x
    m_ref[...] = jnp.concatenate(
        [jnp.maximum(x, 0.0), jnp.zeros_like(x)], axis=1)


def _tc_init(f_bonds, W_i, b_i):
    grid = N_BONDS // _BLK
    return pl.pallas_call(
        _tc_init_body,
        grid=(grid,),
        in_specs=[
            pl.BlockSpec((_BLK, 16), lambda i: (i, 0)),
            pl.BlockSpec((16, H), lambda i: (0, 0)),
            pl.BlockSpec((1, H), lambda i: (0, 0)),
        ],
        out_specs=[
            pl.BlockSpec((_BLK, H), lambda i: (i, 0)),
            pl.BlockSpec((_BLK, GW), lambda i: (i, 0)),
        ],
        out_shape=[
            jax.ShapeDtypeStruct((N_BONDS, H), jnp.float32),
            jax.ShapeDtypeStruct((N_BONDS, GW), jnp.float32),
        ],
    )(f_bonds, W_i, b_i)


def _tc_update_body(t_ref, inp0_ref, wh_ref, bh_ref, m_ref):
    x = jnp.dot(t_ref[...], wh_ref[...],
                preferred_element_type=jnp.float32) + bh_ref[...]
    m = jnp.maximum(inp0_ref[...] + x, 0.0)
    m_ref[...] = jnp.concatenate([m, jnp.zeros_like(m)], axis=1)


def _tc_update(T, inp0, W_h, b_h):
    grid = N_BONDS // _BLK
    return pl.pallas_call(
        _tc_update_body,
        grid=(grid,),
        in_specs=[
            pl.BlockSpec((_BLK, H), lambda i: (i, 0)),
            pl.BlockSpec((_BLK, H), lambda i: (i, 0)),
            pl.BlockSpec((H, H), lambda i: (0, 0)),
            pl.BlockSpec((1, H), lambda i: (0, 0)),
        ],
        out_specs=pl.BlockSpec((_BLK, GW), lambda i: (i, 0)),
        out_shape=jax.ShapeDtypeStruct((N_BONDS, GW), jnp.float32),
    )(T, inp0, W_h, b_h)


_ABLK = 1000
N_MOLS = 256
OUT = 128


def _tc_readout_body(fa_ref, am_ref, mol_ref, wo_ref, bo_ref,
                     wf0_ref, bf0_ref, wf1_ref, bf1_ref, wf2_ref, bf2_ref,
                     out_ref, seg_ref, cnt_ref):
    i = pl.program_id(0)

    @pl.when(i == 0)
    def _init():
        seg_ref[...] = jnp.zeros_like(seg_ref)
        cnt_ref[...] = jnp.zeros_like(cnt_ref)

    a_input = jnp.concatenate([fa_ref[...], am_ref[:, :H]], axis=1)
    hid = jnp.maximum(
        jnp.dot(a_input, wo_ref[...], preferred_element_type=jnp.float32)
        + bo_ref[...], 0.0)
    mol = mol_ref[0, 0, :]
    onehot = (lax.broadcasted_iota(jnp.int32, (N_MOLS, _ABLK), 0)
              == mol[None, :]).astype(jnp.float32)
    seg_ref[...] += jnp.dot(onehot, hid, preferred_element_type=jnp.float32)
    cnt_ref[...] += jnp.sum(onehot, axis=1, keepdims=True)

    @pl.when(i == pl.num_programs(0) - 1)
    def _final():
        cnt = cnt_ref[...]
        mean = jnp.where(cnt > 0.0,
                         seg_ref[...] / jnp.maximum(cnt, 1.0), 0.0)
        h = jnp.maximum(
            jnp.dot(mean, wf0_ref[...], preferred_element_type=jnp.float32)
            + bf0_ref[...], 0.0)
        h = jnp.maximum(
            jnp.dot(h, wf1_ref[...], preferred_element_type=jnp.float32)
            + bf1_ref[...], 0.0)
        out_ref[...] = (jnp.dot(h, wf2_ref[...],
                                preferred_element_type=jnp.float32)
                        + bf2_ref[...])


def _tc_readout(f_atoms, a_message, mol_r, W_o, b_o,
                W_f0, b_f0, W_f1, b_f1, W_f2, b_f2):
    grid = N_ATOMS // _ABLK
    return pl.pallas_call(
        _tc_readout_body,
        grid=(grid,),
        in_specs=[
            pl.BlockSpec((_ABLK, 128), lambda i: (i, 0)),
            pl.BlockSpec((_ABLK, GW), lambda i: (i, 0)),
            pl.BlockSpec((1, 1, _ABLK), lambda i: (i, 0, 0)),
            pl.BlockSpec((128 + H, H), lambda i: (0, 0)),
            pl.BlockSpec((1, H), lambda i: (0, 0)),
            pl.BlockSpec((H, OUT), lambda i: (0, 0)),
            pl.BlockSpec((1, OUT), lambda i: (0, 0)),
            pl.BlockSpec((OUT, OUT), lambda i: (0, 0)),
            pl.BlockSpec((1, OUT), lambda i: (0, 0)),
            pl.BlockSpec((OUT, 1), lambda i: (0, 0)),
            pl.BlockSpec((1, 1), lambda i: (0, 0)),
        ],
        out_specs=pl.BlockSpec((N_MOLS, 1), lambda i: (0, 0)),
        out_shape=jax.ShapeDtypeStruct((N_MOLS, 1), jnp.float32),
        scratch_shapes=[
            pltpu.VMEM((N_MOLS, H), jnp.float32),
            pltpu.VMEM((N_MOLS, 1), jnp.float32),
        ],
    )(f_atoms, a_message, mol_r, W_o, b_o,
      W_f0, b_f0, W_f1, b_f1, W_f2, b_f2)


def kernel(f_atoms, f_bonds, a2b, b2a, b2revb, mol_ids,
           W_i, b_i, W_h, b_h, W_o, b_o,
           W_f0, b_f0, W_f1, b_f1, W_f2, b_f2):
    DEPTH = 4

    a2b_r = jnp.pad(a2b.astype(jnp.int32), ((0, N_ATOMS_P - N_ATOMS), (0, 0))
                    ).reshape(NW, A_CHUNKS, 128)
    b2a_r = jnp.pad(b2a.astype(jnp.int32), (0, N_BONDS_P - N_BONDS)
                    ).reshape(NW, B_CHUNKS, 128)
    b2revb_r = jnp.pad(b2revb.astype(jnp.int32), (0, N_BONDS_P - N_BONDS)
                       ).reshape(NW, B_CHUNKS, 128)
    mol_r = mol_ids.astype(jnp.int32).reshape(N_ATOMS // _ABLK, 1, _ABLK)

    bi_r = b_i.reshape(1, H)
    bh_r = b_h.reshape(1, H)
    bo_r = b_o.reshape(1, H)
    bf0_r = b_f0.reshape(1, OUT)
    bf1_r = b_f1.reshape(1, OUT)
    bf2_r = b_f2.reshape(1, 1)

    inp0, message = _tc_init(f_bonds, W_i, bi_r)
    for _ in range(DEPTH - 1):
        a_message = _sc_gather_sum(message, a2b_r)
        T = _sc_bond_update(a_message, message, b2a_r, b2revb_r)
        message = _tc_update(T, inp0, W_h, bh_r)
    a_message = _sc_gather_sum(message, a2b_r)
    out = _tc_readout(f_atoms, a_message, mol_r, W_o, bo_r,
                      W_f0, bf0_r, W_f1, bf1_r, W_f2, bf2_r)
    return out.reshape(-1)


# speedup vs baseline: 1.0666x; 1.0666x over previous
import jax
import jax.numpy as jnp
from jax import lax
from jax.experimental import pallas as pl
from jax.experimental.pallas import tpu as pltpu
from jax.experimental.pallas import tpu_sc as plsc

NC = 2
NS = 16
NW = NC * NS

N_ATOMS = 10000
N_BONDS = 320000
MAX_NB = 32
H = 64

N_ATOMS_P = 10240
ATOMS_PER_W = N_ATOMS_P // NW
A_CHUNK_ATOMS = 4
A_CHUNKS = ATOMS_PER_W // A_CHUNK_ATOMS

N_BONDS_P = 327680
BONDS_PER_W = N_BONDS_P // NW
B_CHUNK = 64
B_CHUNKS = BONDS_PER_W // B_CHUNK

_SC_MESH = plsc.VectorSubcoreMesh(core_axis_name="c", subcore_axis_name="s")


def _worker_id():
    return lax.axis_index("s") * NC + lax.axis_index("c")


GW = 128


_A_NBUF = 2


def _sc_gather_sum_body(m_hbm, a2b_hbm, out_hbm,
                        idx_v, r0, r1, a0, a1,
                        g0, g1, s0, s1):
    rows = [r0, r1]
    accs = [a0, a1]
    gsem = [g0, g1]
    ssem = [s0, s1]
    wid = _worker_id()
    pltpu.sync_copy(a2b_hbm.at[wid], idx_v)

    for b in range(_A_NBUF):
        pltpu.async_copy(m_hbm.at[idx_v.at[b]], rows[b], gsem[b])

    n_outer = A_CHUNKS // _A_NBUF

    def body(g, carry):
        for b in range(_A_NBUF):
            j = g * _A_NBUF + b
            pltpu.make_async_copy(m_hbm.at[idx_v.at[0]], rows[b],
                                  gsem[b]).wait()

            @pl.when(g > 0)
            def _drain():
                pltpu.make_async_copy(
                    accs[b], out_hbm.at[pl.ds(0, A_CHUNK_ATOMS)],
                    ssem[b]).wait()

            for a in range(A_CHUNK_ATOMS):
                for v in range(H // 16):
                    acc = rows[b][a * MAX_NB, pl.ds(v * 16, 16)]
                    for k in range(1, MAX_NB):
                        acc = acc + rows[b][a * MAX_NB + k, pl.ds(v * 16, 16)]
                    accs[b][a, pl.ds(v * 16, 16)] = acc
            base = wid * ATOMS_PER_W + j * A_CHUNK_ATOMS
            pltpu.async_copy(accs[b], out_hbm.at[pl.ds(base, A_CHUNK_ATOMS)],
                             ssem[b])

            @pl.when(j + _A_NBUF < A_CHUNKS)
            def _next():
                pltpu.async_copy(m_hbm.at[idx_v.at[j + _A_NBUF]], rows[b],
                                 gsem[b])
        return carry

    lax.fori_loop(0, n_outer, body, 0)
    for b in range(_A_NBUF):
        pltpu.make_async_copy(accs[b], out_hbm.at[pl.ds(0, A_CHUNK_ATOMS)],
                              ssem[b]).wait()


def _sc_gather_sum(message, a2b_r):
    k = pl.kernel(
        _sc_gather_sum_body,
        out_type=jax.ShapeDtypeStruct((N_ATOMS_P, GW), jnp.float32),
        mesh=_SC_MESH,
        scratch_types=(
            [pltpu.VMEM((A_CHUNKS, 128), jnp.int32)]
            + [pltpu.VMEM((128, GW), jnp.float32)] * _A_NBUF
            + [pltpu.VMEM((A_CHUNK_ATOMS, GW), jnp.float32)] * _A_NBUF
            + [pltpu.SemaphoreType.DMA] * (2 * _A_NBUF)
        ),
    )
    return k(message, a2b_r)


_B_NBUF = 2


def _sc_bond_update_body(am_hbm, m_hbm, b2a_hbm, b2revb_hbm, t_hbm,
                         idx1_v, idx2_v, am0, am1, rv0, rv1, t0, t1,
                         ga0, ga1, gr0, gr1, st0, st1):
    ams = [am0, am1]
    revs = [rv0, rv1]
    tbufs = [t0, t1]
    gasem = [ga0, ga1]
    grsem = [gr0, gr1]
    stsem = [st0, st1]
    wid = _worker_id()
    pltpu.sync_copy(b2a_hbm.at[wid], idx1_v)
    pltpu.sync_copy(b2revb_hbm.at[wid], idx2_v)

    for b in range(_B_NBUF):
        pltpu.async_copy(am_hbm.at[idx1_v.at[b]], ams[b], gasem[b])
        pltpu.async_copy(m_hbm.at[idx2_v.at[b]], revs[b], grsem[b])

    n_outer = B_CHUNKS // _B_NBUF

    def body(g, carry):
        for b in range(_B_NBUF):
            j = g * _B_NBUF + b
            pltpu.make_async_copy(am_hbm.at[idx1_v.at[0]], ams[b],
                                  gasem[b]).wait()
            pltpu.make_async_copy(m_hbm.at[idx2_v.at[0]], revs[b],
                                  grsem[b]).wait()

            @pl.when(g > 0)
            def _drain():
                pltpu.make_async_copy(tbufs[b], t_hbm.at[pl.ds(0, B_CHUNK)],
                                      stsem[b]).wait()

            for r in range(B_CHUNK):
                for v in range(H // 16):
                    sl = pl.ds(v * 16, 16)
                    tbufs[b][r, sl] = ams[b][r, sl] - revs[b][r, sl]
            base = wid * BONDS_PER_W + j * B_CHUNK
            pltpu.async_copy(tbufs[b], t_hbm.at[pl.ds(base, B_CHUNK)],
                             stsem[b])

            @pl.when(j + _B_NBUF < B_CHUNKS)
            def _next():
                pltpu.async_copy(am_hbm.at[idx1_v.at[j + _B_NBUF]], ams[b],
                                 gasem[b])
                pltpu.async_copy(m_hbm.at[idx2_v.at[j + _B_NBUF]], revs[b],
                                 grsem[b])
        return carry

    lax.fori_loop(0, n_outer, body, 0)
    for b in range(_B_NBUF):
        pltpu.make_async_copy(tbufs[b], t_hbm.at[pl.ds(0, B_CHUNK)],
                              stsem[b]).wait()


def _sc_bond_update(a_message, message, b2a_r, b2revb_r):
    k = pl.kernel(
        _sc_bond_update_body,
        out_type=jax.ShapeDtypeStruct((N_BONDS_P, H), jnp.float32),
        mesh=_SC_MESH,
        scratch_types=(
            [pltpu.VMEM((B_CHUNKS, B_CHUNK), jnp.int32)] * 2
            + [pltpu.VMEM((B_CHUNK, GW), jnp.float32)] * (2 * _B_NBUF)
            + [pltpu.VMEM((B_CHUNK, H), jnp.float32)] * _B_NBUF
            + [pltpu.SemaphoreType.DMA] * (3 * _B_NBUF)
        ),
    )
    return k(a_message, message, b2a_r, b2revb_r)


_BLK = 8000


def _tc_init_body(fb_ref, wi_ref, bi_ref, inp0_ref, m_ref):
    x = jnp.dot(fb_ref[...], wi_ref[...],
                preferred_element_type=jnp.float32) + bi_ref[...]
    inp0_ref[...] = x
    m_ref[...] = jnp.concatenate(
        [jnp.maximum(x, 0.0), jnp.zeros_like(x)], axis=1)


def _tc_init(f_bonds, W_i, b_i):
    grid = N_BONDS // _BLK
    return pl.pallas_call(
        _tc_init_body,
        grid=(grid,),
        in_specs=[
            pl.BlockSpec((_BLK, 16), lambda i: (i, 0)),
            pl.BlockSpec((16, H), lambda i: (0, 0)),
            pl.BlockSpec((1, H), lambda i: (0, 0)),
        ],
        out_specs=[
            pl.BlockSpec((_BLK, H), lambda i: (i, 0)),
            pl.BlockSpec((_BLK, GW), lambda i: (i, 0)),
        ],
        out_shape=[
            jax.ShapeDtypeStruct((N_BONDS, H), jnp.float32),
            jax.ShapeDtypeStruct((N_BONDS, GW), jnp.float32),
        ],
    )(f_bonds, W_i, b_i)


def _tc_update_body(t_ref, inp0_ref, wh_ref, bh_ref, m_ref):
    x = jnp.dot(t_ref[...], wh_ref[...],
                preferred_element_type=jnp.float32) + bh_ref[...]
    m = jnp.maximum(inp0_ref[...] + x, 0.0)
    m_ref[...] = jnp.concatenate([m, jnp.zeros_like(m)], axis=1)


def _tc_update(T, inp0, W_h, b_h):
    grid = N_BONDS // _BLK
    return pl.pallas_call(
        _tc_update_body,
        grid=(grid,),
        in_specs=[
            pl.BlockSpec((_BLK, H), lambda i: (i, 0)),
            pl.BlockSpec((_BLK, H), lambda i: (i, 0)),
            pl.BlockSpec((H, H), lambda i: (0, 0)),
            pl.BlockSpec((1, H), lambda i: (0, 0)),
        ],
        out_specs=pl.BlockSpec((_BLK, GW), lambda i: (i, 0)),
        out_shape=jax.ShapeDtypeStruct((N_BONDS, GW), jnp.float32),
    )(T, inp0, W_h, b_h)


_ABLK = 1000
N_MOLS = 256
OUT = 128


def _tc_readout_body(fa_ref, am_ref, mol_ref, wo_ref, bo_ref,
                     wf0_ref, bf0_ref, wf1_ref, bf1_ref, wf2_ref, bf2_ref,
                     out_ref, seg_ref, cnt_ref):
    i = pl.program_id(0)

    @pl.when(i == 0)
    def _init():
        seg_ref[...] = jnp.zeros_like(seg_ref)
        cnt_ref[...] = jnp.zeros_like(cnt_ref)

    a_input = jnp.concatenate([fa_ref[...], am_ref[:, :H]], axis=1)
    hid = jnp.maximum(
        jnp.dot(a_input, wo_ref[...], preferred_element_type=jnp.float32)
        + bo_ref[...], 0.0)
    mol = mol_ref[0, 0, :]
    onehot = (lax.broadcasted_iota(jnp.int32, (N_MOLS, _ABLK), 0)
              == mol[None, :]).astype(jnp.float32)
    seg_ref[...] += jnp.dot(onehot, hid, preferred_element_type=jnp.float32)
    cnt_ref[...] += jnp.sum(onehot, axis=1, keepdims=True)

    @pl.when(i == pl.num_programs(0) - 1)
    def _final():
        cnt = cnt_ref[...]
        mean = jnp.where(cnt > 0.0,
                         seg_ref[...] / jnp.maximum(cnt, 1.0), 0.0)
        h = jnp.maximum(
            jnp.dot(mean, wf0_ref[...], preferred_element_type=jnp.float32)
            + bf0_ref[...], 0.0)
        h = jnp.maximum(
            jnp.dot(h, wf1_ref[...], preferred_element_type=jnp.float32)
            + bf1_ref[...], 0.0)
        out_ref[...] = (jnp.dot(h, wf2_ref[...],
                                preferred_element_type=jnp.float32)
                        + bf2_ref[...])


def _tc_readout(f_atoms, a_message, mol_r, W_o, b_o,
                W_f0, b_f0, W_f1, b_f1, W_f2, b_f2):
    grid = N_ATOMS // _ABLK
    return pl.pallas_call(
        _tc_readout_body,
        grid=(grid,),
        in_specs=[
            pl.BlockSpec((_ABLK, 128), lambda i: (i, 0)),
            pl.BlockSpec((_ABLK, GW), lambda i: (i, 0)),
            pl.BlockSpec((1, 1, _ABLK), lambda i: (i, 0, 0)),
            pl.BlockSpec((128 + H, H), lambda i: (0, 0)),
            pl.BlockSpec((1, H), lambda i: (0, 0)),
            pl.BlockSpec((H, OUT), lambda i: (0, 0)),
            pl.BlockSpec((1, OUT), lambda i: (0, 0)),
            pl.BlockSpec((OUT, OUT), lambda i: (0, 0)),
            pl.BlockSpec((1, OUT), lambda i: (0, 0)),
            pl.BlockSpec((OUT, 1), lambda i: (0, 0)),
            pl.BlockSpec((1, 1), lambda i: (0, 0)),
        ],
        out_specs=pl.BlockSpec((N_MOLS, 1), lambda i: (0, 0)),
        out_shape=jax.ShapeDtypeStruct((N_MOLS, 1), jnp.float32),
        scratch_shapes=[
            pltpu.VMEM((N_MOLS, H), jnp.float32),
            pltpu.VMEM((N_MOLS, 1), jnp.float32),
        ],
    )(f_atoms, a_message, mol_r, W_o, b_o,
      W_f0, b_f0, W_f1, b_f1, W_f2, b_f2)


def kernel(f_atoms, f_bonds, a2b, b2a, b2revb, mol_ids,
           W_i, b_i, W_h, b_h, W_o, b_o,
           W_f0, b_f0, W_f1, b_f1, W_f2, b_f2):
    DEPTH = 4

    a2b_r = jnp.pad(a2b.astype(jnp.int32), ((0, N_ATOMS_P - N_ATOMS), (0, 0))
                    ).reshape(NW, A_CHUNKS, 128)
    b2a_r = jnp.pad(b2a.astype(jnp.int32), (0, N_BONDS_P - N_BONDS)
                    ).reshape(NW, B_CHUNKS, B_CHUNK)
    b2revb_r = jnp.pad(b2revb.astype(jnp.int32), (0, N_BONDS_P - N_BONDS)
                       ).reshape(NW, B_CHUNKS, B_CHUNK)
    mol_r = mol_ids.astype(jnp.int32).reshape(N_ATOMS // _ABLK, 1, _ABLK)

    bi_r = b_i.reshape(1, H)
    bh_r = b_h.reshape(1, H)
    bo_r = b_o.reshape(1, H)
    bf0_r = b_f0.reshape(1, OUT)
    bf1_r = b_f1.reshape(1, OUT)
    bf2_r = b_f2.reshape(1, 1)

    inp0, message = _tc_init(f_bonds, W_i, bi_r)
    for _ in range(DEPTH - 1):
        a_message = _sc_gather_sum(message, a2b_r)
        T = _sc_bond_update(a_message, message, b2a_r, b2revb_r)
        message = _tc_update(T, inp0, W_h, bh_r)
    a_message = _sc_gather_sum(message, a2b_r)
    out = _tc_readout(f_atoms, a_message, mol_r, W_o, bo_r,
                      W_f0, bf0_r, W_f1, bf1_r, W_f2, bf2_r)
    return out.reshape(-1)
